# baseline (speedup 1.0000x reference)
import functools

import jax
import jax.numpy as jnp
from jax import lax
from jax.experimental import pallas as pl
from jax.experimental.pallas import tpu as pltpu
from jax.experimental.pallas import tpu_sc as plsc

BATCH = 16384
HIDDEN = 64
NROWS = 1000000
PROWS = NROWS // 2
NC = 2
NS = 16
NW = NC * NS
BPW = BATCH // NW
CHUNK = 128
NCHUNK = BPW // CHUNK


TBLK = 2048
TGRID = -(-NROWS // TBLK)


def _tpack_body(eye_ref, tt_ref, o_ref):
    x = tt_ref[...]
    t = lax.dot_general(x, eye_ref[...], (((0,), (0,)), ((), ())),
                        preferred_element_type=jnp.float32)
    t3 = t.reshape(TBLK // 2, 2, HIDDEN)
    o_ref[...] = jnp.concatenate([t3[:, 0, :], t3[:, 1, :]], axis=-1)


_tpack = pl.pallas_call(
    _tpack_body,
    grid=(TGRID,),
    in_specs=[
        pl.BlockSpec((HIDDEN, HIDDEN), lambda n: (0, 0)),
        pl.BlockSpec((HIDDEN, TBLK), lambda n: (0, n)),
    ],
    out_specs=pl.BlockSpec((TBLK // 2, 2 * HIDDEN), lambda n: (n, 0)),
    out_shape=jax.ShapeDtypeStruct((PROWS, 2 * HIDDEN), jnp.float32),
    compiler_params=pltpu.CompilerParams(
        dimension_semantics=("arbitrary",)),
)


_mesh = plsc.VectorSubcoreMesh(core_axis_name="c", subcore_axis_name="s")


@functools.partial(
    pl.kernel,
    mesh=_mesh,
    out_type=[
        jax.ShapeDtypeStruct((BATCH, 2 * HIDDEN), jnp.float32),
        jax.ShapeDtypeStruct((BATCH, 2 * HIDDEN), jnp.float32),
    ],
    scratch_types=[
        pltpu.VMEM((NCHUNK, CHUNK), jnp.int32),
        pltpu.VMEM((NCHUNK, CHUNK), jnp.int32),
        pltpu.VMEM((CHUNK, 2 * HIDDEN), jnp.float32),
        pltpu.VMEM((CHUNK, 2 * HIDDEN), jnp.float32),
        pltpu.SemaphoreType.DMA,
        pltpu.SemaphoreType.DMA,
        pltpu.SemaphoreType.DMA,
    ],
    compiler_params=pltpu.CompilerParams(use_tc_tiling_on_sc=True,
                                         needs_layout_passes=False),
)
def _sc_gather(urow_hbm, irow_hbm, pu_hbm, pi_hbm, uout_hbm, iout_hbm,
               urow_v, irow_v, rbuf0, rbuf1, gsem0, gsem1, wsem):
    wid = lax.axis_index("s") * NC + lax.axis_index("c")
    base = wid * BPW
    pltpu.sync_copy(urow_hbm.at[wid], urow_v)
    pltpu.sync_copy(irow_hbm.at[wid], irow_v)

    chunks = []
    for tblp, rref, out in ((pu_hbm, urow_v, uout_hbm),
                            (pi_hbm, irow_v, iout_hbm)):
        for j in range(NCHUNK):
            chunks.append((tblp, rref, out, j))
    rbufs = (rbuf0, rbuf1)
    gsems = (gsem0, gsem1)

    def fire(slot):
        tblp, rref, _, j = chunks[slot]
        return pltpu.async_copy(tblp.at[rref.at[j]],
                                rbufs[slot % 2], gsems[slot % 2])

    pending = fire(0)
    writes = [None, None]
    for slot in range(len(chunks)):
        _, _, out, j = chunks[slot]
        nxt = None
        if slot + 1 < len(chunks):
            if writes[(slot + 1) % 2] is not None:
                writes[(slot + 1) % 2].wait()
                writes[(slot + 1) % 2] = None
            nxt = fire(slot + 1)
        pending.wait()
        writes[slot % 2] = pltpu.async_copy(
            rbufs[slot % 2], out.at[pl.ds(base + j * CHUNK, CHUNK)], wsem)
        pending = nxt
    writes[0].wait()
    writes[1].wait()


BLK = 2048


def _mlp_body(u_ref, i_ref, up_ref, ip_ref, w1_ref, b1_ref, w2_ref, b2_ref,
              w3_ref, b3_ref, o_ref):
    ug = u_ref[...]
    ig = i_ref[...]
    u = jnp.where(up_ref[...] > 0, ug[:, HIDDEN:], ug[:, :HIDDEN])
    it = jnp.where(ip_ref[...] > 0, ig[:, HIDDEN:], ig[:, :HIDDEN])
    h = jnp.maximum(
        u @ w1_ref[:HIDDEN, :] + it @ w1_ref[HIDDEN:, :] + b1_ref[...], 0.0)
    h = jnp.maximum(h @ w2_ref[...] + b2_ref[...], 0.0)
    o_ref[...] = h @ w3_ref[...] + b3_ref[...]


_mlp = pl.pallas_call(
    _mlp_body,
    grid=(BATCH // BLK,),
    in_specs=[
        pl.BlockSpec((BLK, 2 * HIDDEN), lambda n: (n, 0)),
        pl.BlockSpec((BLK, 2 * HIDDEN), lambda n: (n, 0)),
        pl.BlockSpec((BLK, 1), lambda n: (n, 0)),
        pl.BlockSpec((BLK, 1), lambda n: (n, 0)),
        pl.BlockSpec((2 * HIDDEN, HIDDEN), lambda n: (0, 0)),
        pl.BlockSpec((1, HIDDEN), lambda n: (0, 0)),
        pl.BlockSpec((HIDDEN, HIDDEN // 2), lambda n: (0, 0)),
        pl.BlockSpec((1, HIDDEN // 2), lambda n: (0, 0)),
        pl.BlockSpec((HIDDEN // 2, HIDDEN // 4), lambda n: (0, 0)),
        pl.BlockSpec((1, HIDDEN // 4), lambda n: (0, 0)),
    ],
    out_specs=pl.BlockSpec((BLK, HIDDEN // 4), lambda n: (n, 0)),
    out_shape=jax.ShapeDtypeStruct((BATCH, HIDDEN // 4), jnp.float32),
    compiler_params=pltpu.CompilerParams(
        dimension_semantics=("arbitrary",)),
)


def kernel(user_id, item_id, user_table, item_table, W1, b1, W2, b2, W3, b3):
    uid = user_id.astype(jnp.int32)
    iid = item_id.astype(jnp.int32)
    eye = jnp.eye(HIDDEN, dtype=jnp.float32)
    pu = _tpack(eye, user_table.T)
    pi = _tpack(eye, item_table.T)
    urow = (uid // 2).reshape(NW, NCHUNK, CHUNK)
    irow = (iid // 2).reshape(NW, NCHUNK, CHUNK)
    ug, ig = _sc_gather(urow, irow, pu, pi)
    upar = (uid & 1).astype(jnp.float32).reshape(BATCH, 1)
    ipar = (iid & 1).astype(jnp.float32).reshape(BATCH, 1)
    return _mlp(ug, ig, upar, ipar, W1, b1.reshape(1, -1),
                W2, b2.reshape(1, -1), W3, b3.reshape(1, -1))


# speedup vs baseline: 2.0949x; 2.0949x over previous
import functools

import jax
import jax.numpy as jnp
from jax import lax
from jax.experimental import pallas as pl
from jax.experimental.pallas import tpu as pltpu
from jax.experimental.pallas import tpu_sc as plsc

BATCH = 16384
HIDDEN = 64
NROWS = 1000000
NC = 2
NS = 16
NW = NC * NS
BPW = BATCH // NW
CHUNK = 128
NCHUNK = BPW // CHUNK


TBLK = 4096
HALF = 122 * TBLK
PROWS2 = NROWS - HALF
TGRID = -(-PROWS2 // TBLK)


def _tpack_body(lo_ref, hi_ref, o_ref):
    o_ref[:, :HIDDEN] = lo_ref[...].T
    o_ref[:, HIDDEN:] = hi_ref[...].T


_tpack = pl.pallas_call(
    _tpack_body,
    grid=(TGRID,),
    in_specs=[
        pl.BlockSpec((HIDDEN, TBLK), lambda n: (0, n)),
        pl.BlockSpec((HIDDEN, TBLK), lambda n: (0, n + 122)),
    ],
    out_specs=pl.BlockSpec((TBLK, 2 * HIDDEN), lambda n: (n, 0)),
    out_shape=jax.ShapeDtypeStruct((PROWS2, 2 * HIDDEN), jnp.float32),
    compiler_params=pltpu.CompilerParams(
        dimension_semantics=("arbitrary",)),
)


_mesh = plsc.VectorSubcoreMesh(core_axis_name="c", subcore_axis_name="s")


@functools.partial(
    pl.kernel,
    mesh=_mesh,
    out_type=[
        jax.ShapeDtypeStruct((BATCH, 2 * HIDDEN), jnp.float32),
        jax.ShapeDtypeStruct((BATCH, 2 * HIDDEN), jnp.float32),
    ],
    scratch_types=[
        pltpu.VMEM((NCHUNK, CHUNK), jnp.int32),
        pltpu.VMEM((NCHUNK, CHUNK), jnp.int32),
        pltpu.VMEM((CHUNK, 2 * HIDDEN), jnp.float32),
        pltpu.VMEM((CHUNK, 2 * HIDDEN), jnp.float32),
        pltpu.SemaphoreType.DMA,
        pltpu.SemaphoreType.DMA,
        pltpu.SemaphoreType.DMA,
    ],
    compiler_params=pltpu.CompilerParams(use_tc_tiling_on_sc=True,
                                         needs_layout_passes=False),
)
def _sc_gather(urow_hbm, irow_hbm, pu_hbm, pi_hbm, uout_hbm, iout_hbm,
               urow_v, irow_v, rbuf0, rbuf1, gsem0, gsem1, wsem):
    wid = lax.axis_index("s") * NC + lax.axis_index("c")
    base = wid * BPW
    pltpu.sync_copy(urow_hbm.at[wid], urow_v)
    pltpu.sync_copy(irow_hbm.at[wid], irow_v)

    chunks = []
    for tblp, rref, out in ((pu_hbm, urow_v, uout_hbm),
                            (pi_hbm, irow_v, iout_hbm)):
        for j in range(NCHUNK):
            chunks.append((tblp, rref, out, j))
    rbufs = (rbuf0, rbuf1)
    gsems = (gsem0, gsem1)

    def fire(slot):
        tblp, rref, _, j = chunks[slot]
        return pltpu.async_copy(tblp.at[rref.at[j]],
                                rbufs[slot % 2], gsems[slot % 2])

    pending = fire(0)
    writes = [None, None]
    for slot in range(len(chunks)):
        _, _, out, j = chunks[slot]
        nxt = None
        if slot + 1 < len(chunks):
            if writes[(slot + 1) % 2] is not None:
                writes[(slot + 1) % 2].wait()
                writes[(slot + 1) % 2] = None
            nxt = fire(slot + 1)
        pending.wait()
        writes[slot % 2] = pltpu.async_copy(
            rbufs[slot % 2], out.at[pl.ds(base + j * CHUNK, CHUNK)], wsem)
        pending = nxt
    writes[0].wait()
    writes[1].wait()


BLK = 2048


def _mlp_body(u_ref, i_ref, up_ref, ip_ref, w1_ref, b1_ref, w2_ref, b2_ref,
              w3_ref, b3_ref, o_ref):
    ug = u_ref[...]
    ig = i_ref[...]
    u = jnp.where(up_ref[...] > 0, ug[:, HIDDEN:], ug[:, :HIDDEN])
    it = jnp.where(ip_ref[...] > 0, ig[:, HIDDEN:], ig[:, :HIDDEN])
    h = jnp.maximum(
        u @ w1_ref[:HIDDEN, :] + it @ w1_ref[HIDDEN:, :] + b1_ref[...], 0.0)
    h = jnp.maximum(h @ w2_ref[...] + b2_ref[...], 0.0)
    o_ref[...] = h @ w3_ref[...] + b3_ref[...]


_mlp = pl.pallas_call(
    _mlp_body,
    grid=(BATCH // BLK,),
    in_specs=[
        pl.BlockSpec((BLK, 2 * HIDDEN), lambda n: (n, 0)),
        pl.BlockSpec((BLK, 2 * HIDDEN), lambda n: (n, 0)),
        pl.BlockSpec((BLK, 1), lambda n: (n, 0)),
        pl.BlockSpec((BLK, 1), lambda n: (n, 0)),
        pl.BlockSpec((2 * HIDDEN, HIDDEN), lambda n: (0, 0)),
        pl.BlockSpec((1, HIDDEN), lambda n: (0, 0)),
        pl.BlockSpec((HIDDEN, HIDDEN // 2), lambda n: (0, 0)),
        pl.BlockSpec((1, HIDDEN // 2), lambda n: (0, 0)),
        pl.BlockSpec((HIDDEN // 2, HIDDEN // 4), lambda n: (0, 0)),
        pl.BlockSpec((1, HIDDEN // 4), lambda n: (0, 0)),
    ],
    out_specs=pl.BlockSpec((BLK, HIDDEN // 4), lambda n: (n, 0)),
    out_shape=jax.ShapeDtypeStruct((BATCH, HIDDEN // 4), jnp.float32),
    compiler_params=pltpu.CompilerParams(
        dimension_semantics=("arbitrary",)),
)


def kernel(user_id, item_id, user_table, item_table, W1, b1, W2, b2, W3, b3):
    uid = user_id.astype(jnp.int32)
    iid = item_id.astype(jnp.int32)
    tu = user_table.T
    ti = item_table.T
    pu = _tpack(tu, tu)
    pi = _tpack(ti, ti)
    urow = jnp.where(uid < HALF, uid, uid - HALF).reshape(NW, NCHUNK, CHUNK)
    irow = jnp.where(iid < HALF, iid, iid - HALF).reshape(NW, NCHUNK, CHUNK)
    ug, ig = _sc_gather(urow, irow, pu, pi)
    upar = (uid >= HALF).astype(jnp.float32).reshape(BATCH, 1)
    ipar = (iid >= HALF).astype(jnp.float32).reshape(BATCH, 1)
    return _mlp(ug, ig, upar, ipar, W1, b1.reshape(1, -1),
                W2, b2.reshape(1, -1), W3, b3.reshape(1, -1))


# speedup vs baseline: 2.3601x; 1.1266x over previous
import functools

import jax
import jax.numpy as jnp
from jax import lax
from jax.experimental import pallas as pl
from jax.experimental.pallas import tpu as pltpu
from jax.experimental.pallas import tpu_sc as plsc

BATCH = 16384
HIDDEN = 64
NROWS = 1000000
NC = 2
NS = 16
NW = NC * NS
BPW = BATCH // NW
CHUNK = 128
NCHUNK = BPW // CHUNK


TBLK = 8192
HALF = 61 * TBLK
PROWS2 = NROWS - HALF
TGRID = -(-PROWS2 // TBLK)


def _tpack_body(lo_ref, hi_ref, o_ref):
    o_ref[:, :HIDDEN] = lo_ref[...].T
    o_ref[:, HIDDEN:] = hi_ref[...].T


_tpack = pl.pallas_call(
    _tpack_body,
    grid=(TGRID,),
    in_specs=[
        pl.BlockSpec((HIDDEN, TBLK), lambda n: (0, n)),
        pl.BlockSpec((HIDDEN, TBLK), lambda n: (0, n + 61)),
    ],
    out_specs=pl.BlockSpec((TBLK, 2 * HIDDEN), lambda n: (n, 0)),
    out_shape=jax.ShapeDtypeStruct((PROWS2, 2 * HIDDEN), jnp.float32),
    compiler_params=pltpu.CompilerParams(
        dimension_semantics=("arbitrary",)),
)


_mesh = plsc.VectorSubcoreMesh(core_axis_name="c", subcore_axis_name="s")


@functools.partial(
    pl.kernel,
    mesh=_mesh,
    out_type=jax.ShapeDtypeStruct((BATCH, 2 * HIDDEN), jnp.float32),
    scratch_types=[
        pltpu.VMEM((NCHUNK, CHUNK), jnp.int32),
        pltpu.VMEM((CHUNK, 2 * HIDDEN), jnp.float32),
        pltpu.VMEM((CHUNK, 2 * HIDDEN), jnp.float32),
        pltpu.SemaphoreType.DMA,
        pltpu.SemaphoreType.DMA,
        pltpu.SemaphoreType.DMA,
    ],
    compiler_params=pltpu.CompilerParams(use_tc_tiling_on_sc=True,
                                         needs_layout_passes=False),
)
def _sc_gather(row_hbm, p_hbm, out_hbm, row_v, rbuf0, rbuf1,
               gsem0, gsem1, wsem):
    wid = lax.axis_index("s") * NC + lax.axis_index("c")
    base = wid * BPW
    pltpu.sync_copy(row_hbm.at[wid], row_v)

    chunks = [(p_hbm, row_v, out_hbm, j) for j in range(NCHUNK)]
    rbufs = (rbuf0, rbuf1)
    gsems = (gsem0, gsem1)

    def fire(slot):
        tblp, rref, _, j = chunks[slot]
        return pltpu.async_copy(tblp.at[rref.at[j]],
                                rbufs[slot % 2], gsems[slot % 2])

    pending = fire(0)
    writes = [None, None]
    for slot in range(len(chunks)):
        _, _, out, j = chunks[slot]
        nxt = None
        if slot + 1 < len(chunks):
            if writes[(slot + 1) % 2] is not None:
                writes[(slot + 1) % 2].wait()
                writes[(slot + 1) % 2] = None
            nxt = fire(slot + 1)
        pending.wait()
        writes[slot % 2] = pltpu.async_copy(
            rbufs[slot % 2], out.at[pl.ds(base + j * CHUNK, CHUNK)], wsem)
        pending = nxt
    writes[0].wait()
    writes[1].wait()


BLK = 2048


def _mlp_body(u_ref, i_ref, up_ref, ip_ref, w1_ref, b1_ref, w2_ref, b2_ref,
              w3_ref, b3_ref, o_ref):
    ug = u_ref[...]
    ig = i_ref[...]
    u = jnp.where(up_ref[...] >= HALF, ug[:, HIDDEN:], ug[:, :HIDDEN])
    it = jnp.where(ip_ref[...] >= HALF, ig[:, HIDDEN:], ig[:, :HIDDEN])
    h = jnp.maximum(
        u @ w1_ref[:HIDDEN, :] + it @ w1_ref[HIDDEN:, :] + b1_ref[...], 0.0)
    h = jnp.maximum(h @ w2_ref[...] + b2_ref[...], 0.0)
    o_ref[...] = h @ w3_ref[...] + b3_ref[...]


_mlp = pl.pallas_call(
    _mlp_body,
    grid=(BATCH // BLK,),
    in_specs=[
        pl.BlockSpec((BLK, 2 * HIDDEN), lambda n: (n, 0)),
        pl.BlockSpec((BLK, 2 * HIDDEN), lambda n: (n, 0)),
        pl.BlockSpec((BLK, 1), lambda n: (n, 0)),
        pl.BlockSpec((BLK, 1), lambda n: (n, 0)),
        pl.BlockSpec((2 * HIDDEN, HIDDEN), lambda n: (0, 0)),
        pl.BlockSpec((1, HIDDEN), lambda n: (0, 0)),
        pl.BlockSpec((HIDDEN, HIDDEN // 2), lambda n: (0, 0)),
        pl.BlockSpec((1, HIDDEN // 2), lambda n: (0, 0)),
        pl.BlockSpec((HIDDEN // 2, HIDDEN // 4), lambda n: (0, 0)),
        pl.BlockSpec((1, HIDDEN // 4), lambda n: (0, 0)),
    ],
    out_specs=pl.BlockSpec((BLK, HIDDEN // 4), lambda n: (n, 0)),
    out_shape=jax.ShapeDtypeStruct((BATCH, HIDDEN // 4), jnp.float32),
    compiler_params=pltpu.CompilerParams(
        dimension_semantics=("arbitrary",)),
)


def kernel(user_id, item_id, user_table, item_table, W1, b1, W2, b2, W3, b3):
    uid = user_id.astype(jnp.int32)
    iid = item_id.astype(jnp.int32)
    tu = user_table.T
    ti = item_table.T
    urow = jnp.where(uid < HALF, uid, uid - HALF).reshape(NW, NCHUNK, CHUNK)
    irow = jnp.where(iid < HALF, iid, iid - HALF).reshape(NW, NCHUNK, CHUNK)
    pu = _tpack(tu, tu)
    ug = _sc_gather(urow, pu)
    pi = _tpack(ti, ti)
    ig = _sc_gather(irow, pi)
    return _mlp(ug, ig, uid.reshape(BATCH, 1), iid.reshape(BATCH, 1),
                W1, b1.reshape(1, -1), W2, b2.reshape(1, -1),
                W3, b3.reshape(1, -1))


# speedup vs baseline: 3.2083x; 1.3594x over previous
import functools

import jax
import jax.numpy as jnp
from jax import lax
from jax.experimental import pallas as pl
from jax.experimental.pallas import tpu as pltpu
from jax.experimental.pallas import tpu_sc as plsc

BATCH = 16384
HIDDEN = 64
NROWS = 1000000
NC = 2
NS = 16
NW = NC * NS
BPW = BATCH // NW
CHUNK = 128
NCHUNK = BPW // CHUNK


TBLK = 4096
QTR = 61 * TBLK
N2 = NROWS - 3 * QTR
TGRID = -(-N2 // TBLK)


def _pack2(a_ref, b_ref):
    a = a_ref[...].astype(jnp.bfloat16).T
    b = b_ref[...].astype(jnp.bfloat16).T
    ai = lax.bitcast_convert_type(a, jnp.uint16).astype(jnp.uint32)
    bi = lax.bitcast_convert_type(b, jnp.uint16).astype(jnp.uint32)
    return lax.bitcast_convert_type(ai | (bi << jnp.uint32(16)), jnp.int32)


def _tpack_body(t0_ref, t1_ref, t2_ref, t3_ref, o_ref):
    o_ref[:, :HIDDEN] = _pack2(t0_ref, t1_ref)
    o_ref[:, HIDDEN:] = _pack2(t2_ref, t3_ref)


_tpack = pl.pallas_call(
    _tpack_body,
    grid=(TGRID,),
    in_specs=[
        pl.BlockSpec((HIDDEN, TBLK), lambda n: (0, n)),
        pl.BlockSpec((HIDDEN, TBLK), lambda n: (0, n + 61)),
        pl.BlockSpec((HIDDEN, TBLK), lambda n: (0, n + 122)),
        pl.BlockSpec((HIDDEN, TBLK), lambda n: (0, n + 183)),
    ],
    out_specs=pl.BlockSpec((TBLK, 2 * HIDDEN), lambda n: (n, 0)),
    out_shape=jax.ShapeDtypeStruct((N2, 2 * HIDDEN), jnp.int32),
    compiler_params=pltpu.CompilerParams(
        dimension_semantics=("arbitrary",)),
)


_mesh = plsc.VectorSubcoreMesh(core_axis_name="c", subcore_axis_name="s")


@functools.partial(
    pl.kernel,
    mesh=_mesh,
    out_type=jax.ShapeDtypeStruct((BATCH, 2 * HIDDEN), jnp.int32),
    scratch_types=[
        pltpu.VMEM((NCHUNK, CHUNK), jnp.int32),
        pltpu.VMEM((CHUNK, 2 * HIDDEN), jnp.int32),
        pltpu.VMEM((CHUNK, 2 * HIDDEN), jnp.int32),
        pltpu.SemaphoreType.DMA,
        pltpu.SemaphoreType.DMA,
        pltpu.SemaphoreType.DMA,
    ],
    compiler_params=pltpu.CompilerParams(use_tc_tiling_on_sc=True,
                                         needs_layout_passes=False),
)
def _sc_gather(row_hbm, p_hbm, out_hbm, row_v, rbuf0, rbuf1,
               gsem0, gsem1, wsem):
    wid = lax.axis_index("s") * NC + lax.axis_index("c")
    base = wid * BPW
    pltpu.sync_copy(row_hbm.at[wid], row_v)

    chunks = [(p_hbm, row_v, out_hbm, j) for j in range(NCHUNK)]
    rbufs = (rbuf0, rbuf1)
    gsems = (gsem0, gsem1)

    def fire(slot):
        tblp, rref, _, j = chunks[slot]
        return pltpu.async_copy(tblp.at[rref.at[j]],
                                rbufs[slot % 2], gsems[slot % 2])

    pending = fire(0)
    writes = [None, None]
    for slot in range(len(chunks)):
        _, _, out, j = chunks[slot]
        nxt = None
        if slot + 1 < len(chunks):
            if writes[(slot + 1) % 2] is not None:
                writes[(slot + 1) % 2].wait()
                writes[(slot + 1) % 2] = None
            nxt = fire(slot + 1)
        pending.wait()
        writes[slot % 2] = pltpu.async_copy(
            rbufs[slot % 2], out.at[pl.ds(base + j * CHUNK, CHUNK)], wsem)
        pending = nxt
    writes[0].wait()
    writes[1].wait()


BLK = 2048


def _unpack_lo(w):
    return lax.bitcast_convert_type(
        (w & 0xFFFF).astype(jnp.uint16), jnp.bfloat16).astype(jnp.float32)


def _unpack_hi(w):
    u = lax.shift_right_logical(w.astype(jnp.uint32), jnp.uint32(16))
    return lax.bitcast_convert_type(
        u.astype(jnp.uint16), jnp.bfloat16).astype(jnp.float32)


def _qsel(g, id_col):
    wl = g[:, :HIDDEN]
    wh = g[:, HIDDEN:]
    return jnp.where(
        id_col < QTR, _unpack_lo(wl),
        jnp.where(id_col < 2 * QTR, _unpack_hi(wl),
                  jnp.where(id_col < 3 * QTR, _unpack_lo(wh),
                            _unpack_hi(wh))))


def _mlp_body(u_ref, i_ref, up_ref, ip_ref, w1_ref, b1_ref, w2_ref, b2_ref,
              w3_ref, b3_ref, o_ref):
    u = _qsel(u_ref[...], up_ref[...])
    it = _qsel(i_ref[...], ip_ref[...])
    h = jnp.maximum(
        u @ w1_ref[:HIDDEN, :] + it @ w1_ref[HIDDEN:, :] + b1_ref[...], 0.0)
    h = jnp.maximum(h @ w2_ref[...] + b2_ref[...], 0.0)
    o_ref[...] = h @ w3_ref[...] + b3_ref[...]


_mlp = pl.pallas_call(
    _mlp_body,
    grid=(BATCH // BLK,),
    in_specs=[
        pl.BlockSpec((BLK, 2 * HIDDEN), lambda n: (n, 0)),
        pl.BlockSpec((BLK, 2 * HIDDEN), lambda n: (n, 0)),
        pl.BlockSpec((BLK, 1), lambda n: (n, 0)),
        pl.BlockSpec((BLK, 1), lambda n: (n, 0)),
        pl.BlockSpec((2 * HIDDEN, HIDDEN), lambda n: (0, 0)),
        pl.BlockSpec((1, HIDDEN), lambda n: (0, 0)),
        pl.BlockSpec((HIDDEN, HIDDEN // 2), lambda n: (0, 0)),
        pl.BlockSpec((1, HIDDEN // 2), lambda n: (0, 0)),
        pl.BlockSpec((HIDDEN // 2, HIDDEN // 4), lambda n: (0, 0)),
        pl.BlockSpec((1, HIDDEN // 4), lambda n: (0, 0)),
    ],
    out_specs=pl.BlockSpec((BLK, HIDDEN // 4), lambda n: (n, 0)),
    out_shape=jax.ShapeDtypeStruct((BATCH, HIDDEN // 4), jnp.float32),
    compiler_params=pltpu.CompilerParams(
        dimension_semantics=("arbitrary",)),
)


def kernel(user_id, item_id, user_table, item_table, W1, b1, W2, b2, W3, b3):
    uid = user_id.astype(jnp.int32)
    iid = item_id.astype(jnp.int32)
    tu = user_table.T
    ti = item_table.T
    urow = (uid - jnp.minimum(uid // QTR, 3) * QTR).reshape(NW, NCHUNK, CHUNK)
    irow = (iid - jnp.minimum(iid // QTR, 3) * QTR).reshape(NW, NCHUNK, CHUNK)
    pu = _tpack(tu, tu, tu, tu)
    ug = _sc_gather(urow, pu)
    pi = _tpack(ti, ti, ti, ti)
    ig = _sc_gather(irow, pi)
    return _mlp(ug, ig, uid.reshape(BATCH, 1), iid.reshape(BATCH, 1),
                W1, b1.reshape(1, -1), W2, b2.reshape(1, -1),
                W3, b3.reshape(1, -1))


# speedup vs baseline: 3.4115x; 1.0633x over previous
import functools

import jax
import jax.numpy as jnp
from jax import lax
from jax.experimental import pallas as pl
from jax.experimental.pallas import tpu as pltpu
from jax.experimental.pallas import tpu_sc as plsc

BATCH = 16384
HIDDEN = 64
NROWS = 1000000
NC = 2
NS = 16
NW = NC * NS
BPW = BATCH // NW
CHUNK = 128
NCHUNK = BPW // CHUNK


TBLK = 8192
QTR = 30 * TBLK
N2 = NROWS - 3 * QTR
TGRID = -(-N2 // TBLK)


def _pack2(a_ref, b_ref):
    a = a_ref[...].astype(jnp.bfloat16).T
    b = b_ref[...].astype(jnp.bfloat16).T
    ai = lax.bitcast_convert_type(a, jnp.uint16).astype(jnp.uint32)
    bi = lax.bitcast_convert_type(b, jnp.uint16).astype(jnp.uint32)
    return lax.bitcast_convert_type(ai | (bi << jnp.uint32(16)), jnp.int32)


def _tpack_body(t0_ref, t1_ref, t2_ref, t3_ref, o_ref):
    o_ref[:, :HIDDEN] = _pack2(t0_ref, t1_ref)
    o_ref[:, HIDDEN:] = _pack2(t2_ref, t3_ref)


_tpack = pl.pallas_call(
    _tpack_body,
    grid=(TGRID,),
    in_specs=[
        pl.BlockSpec((HIDDEN, TBLK), lambda n: (0, n)),
        pl.BlockSpec((HIDDEN, TBLK), lambda n: (0, n + 30)),
        pl.BlockSpec((HIDDEN, TBLK), lambda n: (0, n + 60)),
        pl.BlockSpec((HIDDEN, TBLK), lambda n: (0, n + 90)),
    ],
    out_specs=pl.BlockSpec((TBLK, 2 * HIDDEN), lambda n: (n, 0)),
    out_shape=jax.ShapeDtypeStruct((N2, 2 * HIDDEN), jnp.int32),
    compiler_params=pltpu.CompilerParams(
        dimension_semantics=("arbitrary",)),
)


_mesh = plsc.VectorSubcoreMesh(core_axis_name="c", subcore_axis_name="s")


@functools.partial(
    pl.kernel,
    mesh=_mesh,
    out_type=jax.ShapeDtypeStruct((BATCH, 2 * HIDDEN), jnp.int32),
    scratch_types=[
        pltpu.VMEM((NCHUNK, CHUNK), jnp.int32),
        pltpu.VMEM((CHUNK, 2 * HIDDEN), jnp.int32),
        pltpu.VMEM((CHUNK, 2 * HIDDEN), jnp.int32),
        pltpu.SemaphoreType.DMA,
        pltpu.SemaphoreType.DMA,
        pltpu.SemaphoreType.DMA,
    ],
    compiler_params=pltpu.CompilerParams(use_tc_tiling_on_sc=True,
                                         needs_layout_passes=False),
)
def _sc_gather(row_hbm, p_hbm, out_hbm, row_v, rbuf0, rbuf1,
               gsem0, gsem1, wsem):
    wid = lax.axis_index("s") * NC + lax.axis_index("c")
    base = wid * BPW
    pltpu.sync_copy(row_hbm.at[wid], row_v)

    chunks = [(p_hbm, row_v, out_hbm, j) for j in range(NCHUNK)]
    rbufs = (rbuf0, rbuf1)
    gsems = (gsem0, gsem1)

    def fire(slot):
        tblp, rref, _, j = chunks[slot]
        return pltpu.async_copy(tblp.at[rref.at[j]],
                                rbufs[slot % 2], gsems[slot % 2])

    pending = fire(0)
    writes = [None, None]
    for slot in range(len(chunks)):
        _, _, out, j = chunks[slot]
        nxt = None
        if slot + 1 < len(chunks):
            if writes[(slot + 1) % 2] is not None:
                writes[(slot + 1) % 2].wait()
                writes[(slot + 1) % 2] = None
            nxt = fire(slot + 1)
        pending.wait()
        writes[slot % 2] = pltpu.async_copy(
            rbufs[slot % 2], out.at[pl.ds(base + j * CHUNK, CHUNK)], wsem)
        pending = nxt
    writes[0].wait()
    writes[1].wait()


BLK = 4096


def _unpack_lo(w):
    return lax.bitcast_convert_type(
        (w & 0xFFFF).astype(jnp.uint16), jnp.bfloat16).astype(jnp.float32)


def _unpack_hi(w):
    u = lax.shift_right_logical(w.astype(jnp.uint32), jnp.uint32(16))
    return lax.bitcast_convert_type(
        u.astype(jnp.uint16), jnp.bfloat16).astype(jnp.float32)


def _qsel(g, id_col):
    w = jnp.where(id_col < 2 * QTR, g[:, :HIDDEN], g[:, HIDDEN:])
    odd = (id_col >= QTR) & (id_col < 2 * QTR) | (id_col >= 3 * QTR)
    return jnp.where(odd, _unpack_hi(w), _unpack_lo(w))


def _mlp_body(u_ref, i_ref, up_ref, ip_ref, w1_ref, b1_ref, w2_ref, b2_ref,
              w3_ref, b3_ref, o_ref):
    u = _qsel(u_ref[...], up_ref[...])
    it = _qsel(i_ref[...], ip_ref[...])
    h = jnp.maximum(
        u @ w1_ref[:HIDDEN, :] + it @ w1_ref[HIDDEN:, :] + b1_ref[...], 0.0)
    h = jnp.maximum(h @ w2_ref[...] + b2_ref[...], 0.0)
    o_ref[...] = h @ w3_ref[...] + b3_ref[...]


_mlp = pl.pallas_call(
    _mlp_body,
    grid=(BATCH // BLK,),
    in_specs=[
        pl.BlockSpec((BLK, 2 * HIDDEN), lambda n: (n, 0)),
        pl.BlockSpec((BLK, 2 * HIDDEN), lambda n: (n, 0)),
        pl.BlockSpec((BLK, 1), lambda n: (n, 0)),
        pl.BlockSpec((BLK, 1), lambda n: (n, 0)),
        pl.BlockSpec((2 * HIDDEN, HIDDEN), lambda n: (0, 0)),
        pl.BlockSpec((1, HIDDEN), lambda n: (0, 0)),
        pl.BlockSpec((HIDDEN, HIDDEN // 2), lambda n: (0, 0)),
        pl.BlockSpec((1, HIDDEN // 2), lambda n: (0, 0)),
        pl.BlockSpec((HIDDEN // 2, HIDDEN // 4), lambda n: (0, 0)),
        pl.BlockSpec((1, HIDDEN // 4), lambda n: (0, 0)),
    ],
    out_specs=pl.BlockSpec((BLK, HIDDEN // 4), lambda n: (n, 0)),
    out_shape=jax.ShapeDtypeStruct((BATCH, HIDDEN // 4), jnp.float32),
    compiler_params=pltpu.CompilerParams(
        dimension_semantics=("arbitrary",)),
)


def kernel(user_id, item_id, user_table, item_table, W1, b1, W2, b2, W3, b3):
    uid = user_id.astype(jnp.int32)
    iid = item_id.astype(jnp.int32)
    tu = user_table.T
    ti = item_table.T
    urow = (uid - jnp.minimum(uid // QTR, 3) * QTR).reshape(NW, NCHUNK, CHUNK)
    irow = (iid - jnp.minimum(iid // QTR, 3) * QTR).reshape(NW, NCHUNK, CHUNK)
    pu = _tpack(tu, tu, tu, tu)
    ug = _sc_gather(urow, pu)
    pi = _tpack(ti, ti, ti, ti)
    ig = _sc_gather(irow, pi)
    return _mlp(ug, ig, uid.reshape(BATCH, 1), iid.reshape(BATCH, 1),
                W1, b1.reshape(1, -1), W2, b2.reshape(1, -1),
                W3, b3.reshape(1, -1))
